# Initial kernel scaffold; baseline (speedup 1.0000x reference)
import functools

import jax
import jax.numpy as jnp
from jax import lax
from jax.experimental import pallas as pl
from jax.experimental.pallas import tpu as pltpu
from jax.experimental.pallas import tpu_sc as plsc

VOCAB = 1000000
EMBED_DIM = 16
BATCH = 16384
SEQ = 200

N = BATCH * SEQ
NUM_WORKERS = 32
ROWS_PER_WORKER = N // NUM_WORKERS
CHUNK = 2048
NUM_CHUNKS = ROWS_PER_WORKER // CHUNK


def _sc_gather(x_flat, table):
    mesh = plsc.VectorSubcoreMesh(core_axis_name="c", subcore_axis_name="s")

    @functools.partial(
        pl.kernel,
        mesh=mesh,
        out_type=jax.ShapeDtypeStruct((N, EMBED_DIM), jnp.float32),
        scratch_types=[
            pltpu.VMEM((CHUNK,), jnp.int32),
            pltpu.VMEM((CHUNK, EMBED_DIM), jnp.float32),
            pltpu.SemaphoreType.DMA,
        ],
    )
    def k(idx_hbm, table_hbm, out_hbm, idx_v, rows_v, sem):
        wid = lax.axis_index("s") * 2 + lax.axis_index("c")
        base = wid * ROWS_PER_WORKER

        def body(i, _):
            off = base + i * CHUNK
            pltpu.sync_copy(idx_hbm.at[pl.ds(off, CHUNK)], idx_v)
            pltpu.async_copy(table_hbm.at[idx_v], rows_v, sem).wait()
            pltpu.sync_copy(rows_v, out_hbm.at[pl.ds(off, CHUNK)])
            return 0

        lax.fori_loop(0, NUM_CHUNKS, body, 0)

    return k(x_flat, table)


def kernel(x, table):
    x_flat = x.reshape(-1).astype(jnp.int32)
    out = _sc_gather(x_flat, table)
    return out.reshape(BATCH, SEQ, EMBED_DIM)


# speedup vs baseline: 2.4903x; 2.4903x over previous
import functools

import jax
import jax.numpy as jnp
from jax import lax
from jax.experimental import pallas as pl
from jax.experimental.pallas import tpu as pltpu
from jax.experimental.pallas import tpu_sc as plsc

VOCAB = 1000000
EMBED_DIM = 16
BATCH = 16384
SEQ = 200

N = BATCH * SEQ
NUM_WORKERS = 32
ROWS_PER_WORKER = N // NUM_WORKERS
CHUNK = 2048
NUM_CHUNKS = ROWS_PER_WORKER // CHUNK


def _sc_gather(x_flat, table):
    mesh = plsc.VectorSubcoreMesh(core_axis_name="c", subcore_axis_name="s")

    @functools.partial(
        pl.kernel,
        mesh=mesh,
        out_type=jax.ShapeDtypeStruct((N, EMBED_DIM), jnp.float32),
        scratch_types=[
            pltpu.VMEM((CHUNK,), jnp.int32),
            pltpu.VMEM((CHUNK, EMBED_DIM), jnp.float32),
            pltpu.SemaphoreType.DMA,
        ],
        compiler_params=pltpu.CompilerParams(use_tc_tiling_on_sc=False),
    )
    def k(idx_hbm, table_hbm, out_hbm, idx_v, rows_v, sem):
        wid = lax.axis_index("s") * 2 + lax.axis_index("c")
        base = wid * ROWS_PER_WORKER

        def body(i, _):
            off = base + i * CHUNK
            pltpu.sync_copy(idx_hbm.at[pl.ds(off, CHUNK)], idx_v)
            pltpu.async_copy(table_hbm.at[idx_v], rows_v, sem).wait()
            pltpu.sync_copy(rows_v, out_hbm.at[pl.ds(off, CHUNK)])
            return 0

        lax.fori_loop(0, NUM_CHUNKS, body, 0)

    return k(x_flat, table)


def kernel(x, table):
    x_flat = x.reshape(-1).astype(jnp.int32)
    out = _sc_gather(x_flat, table)
    return out.reshape(BATCH, SEQ, EMBED_DIM)


# speedup vs baseline: 2.5662x; 1.0305x over previous
import functools

import jax
import jax.numpy as jnp
from jax import lax
from jax.experimental import pallas as pl
from jax.experimental.pallas import tpu as pltpu
from jax.experimental.pallas import tpu_sc as plsc

VOCAB = 1000000
EMBED_DIM = 16
BATCH = 16384
SEQ = 200

N = BATCH * SEQ
NUM_WORKERS = 32
ROWS_PER_WORKER = N // NUM_WORKERS
CHUNK = 2048
NUM_CHUNKS = ROWS_PER_WORKER // CHUNK
NUM_PAIRS = NUM_CHUNKS // 2


def _sc_gather(x_flat, table):
    mesh = plsc.VectorSubcoreMesh(core_axis_name="c", subcore_axis_name="s")

    @functools.partial(
        pl.kernel,
        mesh=mesh,
        out_type=jax.ShapeDtypeStruct((N, EMBED_DIM), jnp.float32),
        scratch_types=[
            pltpu.VMEM((2, CHUNK), jnp.int32),
            pltpu.VMEM((2, CHUNK, EMBED_DIM), jnp.float32),
            pltpu.SemaphoreType.DMA,
            pltpu.SemaphoreType.DMA,
        ],
        compiler_params=pltpu.CompilerParams(use_tc_tiling_on_sc=False),
    )
    def k(idx_hbm, table_hbm, out_hbm, idx_v, rows_v, sem0, sem1):
        wid = lax.axis_index("s") * 2 + lax.axis_index("c")
        base = wid * ROWS_PER_WORKER
        sems = (sem0, sem1)

        def load_and_gather(chunk, b):
            off = base + chunk * CHUNK
            pltpu.sync_copy(idx_hbm.at[pl.ds(off, CHUNK)], idx_v.at[b])
            pltpu.async_copy(table_hbm.at[idx_v.at[b]], rows_v.at[b], sems[b])

        def wait_and_store(chunk, b):
            off = base + chunk * CHUNK
            pltpu.make_async_copy(
                table_hbm.at[idx_v.at[b]], rows_v.at[b], sems[b]
            ).wait()
            pltpu.sync_copy(rows_v.at[b], out_hbm.at[pl.ds(off, CHUNK)])

        load_and_gather(0, 0)

        def body(g, _):
            i0 = 2 * g
            load_and_gather(i0 + 1, 1)
            wait_and_store(i0, 0)

            @pl.when(g < NUM_PAIRS - 1)
            def _():
                load_and_gather(i0 + 2, 0)

            wait_and_store(i0 + 1, 1)
            return 0

        lax.fori_loop(0, NUM_PAIRS, body, 0)

    return k(x_flat, table)


def kernel(x, table):
    x_flat = x.reshape(-1).astype(jnp.int32)
    out = _sc_gather(x_flat, table)
    return out.reshape(BATCH, SEQ, EMBED_DIM)


# speedup vs baseline: 12.3679x; 4.8195x over previous
import functools

import jax
import jax.numpy as jnp
from jax import lax
from jax.experimental import pallas as pl
from jax.experimental.pallas import tpu as pltpu
from jax.experimental.pallas import tpu_sc as plsc

VOCAB = 1000000
D = 16
BATCH = 16384
SEQ = 200

VMAIN = 999936
W1 = 1536
NCH1 = VMAIN // W1
RB1 = W1 * D // 128
EXTRA1 = NCH1 - 32 * (NCH1 // 32)

NUNIT = (BATCH // 128) * (SEQ // 8)
UPT = NUNIT // 32
NPAIR = UPT // 2


def _mesh():
    return plsc.VectorSubcoreMesh(core_axis_name="c", subcore_axis_name="s")


def _transpose_call(tab_t, tail_lin):
    @functools.partial(
        pl.kernel,
        mesh=_mesh(),
        out_type=jax.ShapeDtypeStruct((125000, 128), jnp.float32),
        scratch_types=[
            pltpu.VMEM((16, W1 + 1), jnp.float32),
            pltpu.VMEM((16, W1 + 1), jnp.float32),
            pltpu.VMEM((RB1, 128), jnp.float32),
            pltpu.VMEM((RB1, 128), jnp.float32),
            pltpu.VMEM((8, 128), jnp.float32),
            pltpu.SemaphoreType.DMA,
            pltpu.SemaphoreType.DMA,
            pltpu.SemaphoreType.DMA,
            pltpu.SemaphoreType.DMA,
        ],
        compiler_params=pltpu.CompilerParams(
            use_tc_tiling_on_sc=True, needs_layout_passes=False),
    )
    def k(tab_hbm, tail_hbm, out_hbm, in0, in1, ob0, ob1, tbuf,
          si0, si1, so0, so1):
        wid = lax.axis_index("s") * 2 + lax.axis_index("c")
        n = jnp.where(wid < EXTRA1, NCH1 // 32 + 1, NCH1 // 32)
        iota = lax.iota(jnp.int32, 16)

        def in_descs(c, buf, sem):
            off = c * W1
            d1 = pltpu.make_async_copy(
                tab_hbm.at[pl.ds(0, 8), pl.ds(off, W1)],
                buf.at[pl.ds(0, 8), pl.ds(0, W1)], sem)
            d2 = pltpu.make_async_copy(
                tab_hbm.at[pl.ds(8, 8), pl.ds(off, W1)],
                buf.at[pl.ds(8, 8), pl.ds(0, W1)], sem)
            return d1, d2

        def start_in(c, buf, sem):
            d1, d2 = in_descs(c, buf, sem)
            d1.start()
            d2.start()

        def wait_in(c, buf, sem):
            d1, d2 = in_descs(c, buf, sem)
            d1.wait()
            d2.wait()

        def transpose_chunk(buf, ob):
            @plsc.parallel_loop(0, W1, unroll=8)
            def _(v):
                val = plsc.load_gather(
                    buf, [iota, jnp.full((16,), v, jnp.int32)])
                ob[v >> 3, pl.ds((v & 7) * 16, 16)] = val

        def out_desc(c, ob, sem):
            return pltpu.make_async_copy(
                ob, out_hbm.at[pl.ds(c * RB1, RB1)], sem)

        def do_chunk(i, c, buf, ob, sem_o):
            wait_in(c, buf, si0 if buf is in0 else si1)
            @pl.when(i >= 2)
            def _():
                out_desc(0, ob, sem_o).wait()
            transpose_chunk(buf, ob)
            out_desc(c, ob, sem_o).start()

        start_in(wid, in0, si0)

        def body(g, _):
            i0, i1 = 2 * g, 2 * g + 1
            c0 = wid + 32 * i0
            c1 = wid + 32 * i1

            @pl.when(i1 < n)
            def _():
                start_in(c1, in1, si1)

            @pl.when(i0 < n)
            def _():
                do_chunk(i0, c0, in0, ob0, so0)

            @pl.when(i0 + 2 < n)
            def _():
                start_in(c0 + 64, in0, si0)

            @pl.when(i1 < n)
            def _():
                do_chunk(i1, c1, in1, ob1, so1)
            return 0

        lax.fori_loop(0, (NCH1 // 32 + 2) // 2, body, 0)
        out_desc(0, ob0, so0).wait()
        out_desc(0, ob1, so1).wait()

        @pl.when(wid == 31)
        def _():
            pltpu.sync_copy(tail_hbm, tbuf)
            pltpu.sync_copy(tbuf, out_hbm.at[pl.ds(VMAIN * D // 128, 8)])

    return k(tab_t, tail_lin)


def _gather_call(x4, tbl_rm):
    @functools.partial(
        pl.kernel,
        mesh=_mesh(),
        out_type=jax.ShapeDtypeStruct((200, 2, 128, 8, 128), jnp.float32),
        scratch_types=[
            pltpu.VMEM((1024,), jnp.int32),
            pltpu.VMEM((1024,), jnp.int32),
            pltpu.VMEM((1024, 16), jnp.float32),
            pltpu.VMEM((1024, 16), jnp.float32),
            pltpu.VMEM((8, 2, 8, 129), jnp.float32),
            pltpu.VMEM((8, 2, 8, 129), jnp.float32),
            pltpu.SemaphoreType.DMA,
            pltpu.SemaphoreType.DMA,
            pltpu.SemaphoreType.DMA,
            pltpu.SemaphoreType.DMA,
        ],
        compiler_params=pltpu.CompilerParams(
            use_tc_tiling_on_sc=False, needs_layout_passes=False),
    )
    def k(x4_hbm, tbl_hbm, out_hbm, ix0, ix1, rw0, rw1, tr0, tr1,
          sg0, sg1, ss0, ss1):
        wid = lax.axis_index("s") * 2 + lax.axis_index("c")
        iota = lax.iota(jnp.int32, 16)
        e8 = iota >> 3
        em = iota & 7

        def stage_and_fire(u, ix, rw, sg):
            unit = u * 32 + wid
            st = unit >> 7
            bt = unit & 127
            pltpu.sync_copy(x4_hbm.at[st, bt], ix)
            pltpu.make_async_copy(tbl_hbm.at[ix], rw, sg).start()

        def wait_gather(ix, rw, sg):
            pltpu.make_async_copy(tbl_hbm.at[ix], rw, sg).wait()

        def transpose_unit(rw, tr):
            for sr in range(8):
                srv = jnp.full((16,), sr, jnp.int32)

                @plsc.parallel_loop(0, 128, unroll=8)
                def _(bc):
                    val = rw[sr * 128 + bc, :]
                    plsc.store_scatter(
                        tr, [srv, e8, em, jnp.full((16,), bc, jnp.int32)],
                        val)

        def fire_stores(u, tr, ss):
            unit = u * 32 + wid
            st = unit >> 7
            bt = unit & 127
            for sr in range(8):
                s = st * 8 + sr
                for et in range(2):
                    pltpu.make_async_copy(
                        tr.at[sr, et, :, pl.ds(0, 128)],
                        out_hbm.at[s, et, bt], ss).start()

        def drain_stores(rw, ss):
            pltpu.make_async_copy(tbl_hbm.at[pl.ds(0, 1024)], rw, ss).wait()

        stage_and_fire(0, ix0, rw0, sg0)

        def body(g, _):
            u0, u1 = 2 * g, 2 * g + 1
            stage_and_fire(u1, ix1, rw1, sg1)
            wait_gather(ix0, rw0, sg0)

            @pl.when(g > 0)
            def _():
                drain_stores(rw1, ss0)
            transpose_unit(rw0, tr0)
            fire_stores(u0, tr0, ss0)

            @pl.when(g < NPAIR - 1)
            def _():
                stage_and_fire(u0 + 2, ix0, rw0, sg0)
            wait_gather(ix1, rw1, sg1)

            @pl.when(g > 0)
            def _():
                drain_stores(rw0, ss1)
            transpose_unit(rw1, tr1)
            fire_stores(u1, tr1, ss1)
            return 0

        lax.fori_loop(0, NPAIR, body, 0)
        drain_stores(rw0, ss0)
        drain_stores(rw1, ss1)

    return k(x4, tbl_rm)


def kernel(x, table):
    x = x.astype(jnp.int32)
    tab_t = table.T
    tail_lin = table[VMAIN:, :].reshape(8, 128)
    x4 = (x.T.reshape(25, 8, 128, 128)
          .transpose(0, 2, 1, 3).reshape(25, 128, 1024))
    tbl_lin = _transpose_call(tab_t, tail_lin)
    tbl_rm = tbl_lin.reshape(VOCAB, D)
    o5 = _gather_call(x4, tbl_rm)
    return o5.transpose(2, 4, 0, 1, 3).reshape(BATCH, SEQ, D)


# speedup vs baseline: 13.2957x; 1.0750x over previous
import functools

import jax
import jax.numpy as jnp
from jax import lax
from jax.experimental import pallas as pl
from jax.experimental.pallas import tpu as pltpu
from jax.experimental.pallas import tpu_sc as plsc

VOCAB = 1000000
D = 16
BATCH = 16384
SEQ = 200

VMAIN = 999936
W1 = 1536
NCH1 = VMAIN // W1
RB1 = W1 * D // 128
EXTRA1 = NCH1 - 32 * (NCH1 // 32)

NUNIT = (BATCH // 128) * (SEQ // 8)
UPT = NUNIT // 32
NPAIR = UPT // 2


def _mesh():
    return plsc.VectorSubcoreMesh(core_axis_name="c", subcore_axis_name="s")


def _transpose_call(tab_t, tail_lin):
    @functools.partial(
        pl.kernel,
        mesh=_mesh(),
        out_type=jax.ShapeDtypeStruct((125000, 128), jnp.float32),
        scratch_types=[
            pltpu.VMEM((16, W1 + 1), jnp.float32),
            pltpu.VMEM((16, W1 + 1), jnp.float32),
            pltpu.VMEM((RB1, 128), jnp.float32),
            pltpu.VMEM((RB1, 128), jnp.float32),
            pltpu.VMEM((8, 128), jnp.float32),
            pltpu.SemaphoreType.DMA,
            pltpu.SemaphoreType.DMA,
            pltpu.SemaphoreType.DMA,
            pltpu.SemaphoreType.DMA,
        ],
        compiler_params=pltpu.CompilerParams(
            use_tc_tiling_on_sc=True, needs_layout_passes=False),
    )
    def k(tab_hbm, tail_hbm, out_hbm, in0, in1, ob0, ob1, tbuf,
          si0, si1, so0, so1):
        wid = lax.axis_index("s") * 2 + lax.axis_index("c")
        n = jnp.where(wid < EXTRA1, NCH1 // 32 + 1, NCH1 // 32)
        iota = lax.iota(jnp.int32, 16)

        def in_descs(c, buf, sem):
            off = c * W1
            d1 = pltpu.make_async_copy(
                tab_hbm.at[pl.ds(0, 8), pl.ds(off, W1)],
                buf.at[pl.ds(0, 8), pl.ds(0, W1)], sem)
            d2 = pltpu.make_async_copy(
                tab_hbm.at[pl.ds(8, 8), pl.ds(off, W1)],
                buf.at[pl.ds(8, 8), pl.ds(0, W1)], sem)
            return d1, d2

        def start_in(c, buf, sem):
            d1, d2 = in_descs(c, buf, sem)
            d1.start()
            d2.start()

        def wait_in(c, buf, sem):
            d1, d2 = in_descs(c, buf, sem)
            d1.wait()
            d2.wait()

        def transpose_chunk(buf, ob):
            @plsc.parallel_loop(0, W1, unroll=8,
                                carry=jnp.zeros((16,), jnp.int32))
            def _(v, colv):
                val = plsc.load_gather(buf, [iota, colv])
                ob[v >> 3, pl.ds((v & 7) * 16, 16)] = val
                return colv + 1

        def out_desc(c, ob, sem):
            return pltpu.make_async_copy(
                ob, out_hbm.at[pl.ds(c * RB1, RB1)], sem)

        def do_chunk(i, c, buf, ob, sem_o):
            wait_in(c, buf, si0 if buf is in0 else si1)
            @pl.when(i >= 2)
            def _():
                out_desc(0, ob, sem_o).wait()
            transpose_chunk(buf, ob)
            out_desc(c, ob, sem_o).start()

        start_in(wid, in0, si0)

        def body(g, _):
            i0, i1 = 2 * g, 2 * g + 1
            c0 = wid + 32 * i0
            c1 = wid + 32 * i1

            @pl.when(i1 < n)
            def _():
                start_in(c1, in1, si1)

            @pl.when(i0 < n)
            def _():
                do_chunk(i0, c0, in0, ob0, so0)

            @pl.when(i0 + 2 < n)
            def _():
                start_in(c0 + 64, in0, si0)

            @pl.when(i1 < n)
            def _():
                do_chunk(i1, c1, in1, ob1, so1)
            return 0

        lax.fori_loop(0, (NCH1 // 32 + 2) // 2, body, 0)
        out_desc(0, ob0, so0).wait()
        out_desc(0, ob1, so1).wait()

        @pl.when(wid == 31)
        def _():
            pltpu.sync_copy(tail_hbm, tbuf)
            pltpu.sync_copy(tbuf, out_hbm.at[pl.ds(VMAIN * D // 128, 8)])

    return k(tab_t, tail_lin)


def _gather_call(x4, tbl_rm):
    @functools.partial(
        pl.kernel,
        mesh=_mesh(),
        out_type=jax.ShapeDtypeStruct((200, 2, 128, 8, 128), jnp.float32),
        scratch_types=[
            pltpu.VMEM((1024,), jnp.int32),
            pltpu.VMEM((1024,), jnp.int32),
            pltpu.VMEM((1024, 16), jnp.float32),
            pltpu.VMEM((1024, 16), jnp.float32),
            pltpu.VMEM((128, 129), jnp.float32),
            pltpu.VMEM((128, 129), jnp.float32),
            pltpu.SemaphoreType.DMA,
            pltpu.SemaphoreType.DMA,
            pltpu.SemaphoreType.DMA,
            pltpu.SemaphoreType.DMA,
        ],
        compiler_params=pltpu.CompilerParams(
            use_tc_tiling_on_sc=False, needs_layout_passes=False),
    )
    def k(x4_hbm, tbl_hbm, out_hbm, ix0, ix1, rw0, rw1, tr0, tr1,
          sg0, sg1, ss0, ss1):
        wid = lax.axis_index("s") * 2 + lax.axis_index("c")
        iota = lax.iota(jnp.int32, 16)

        def stage_and_fire(u, ix, rw, sg):
            unit = u * 32 + wid
            st = unit >> 7
            bt = unit & 127
            pltpu.sync_copy(x4_hbm.at[st, bt], ix)
            pltpu.make_async_copy(tbl_hbm.at[ix], rw, sg).start()

        def wait_gather(ix, rw, sg):
            pltpu.make_async_copy(tbl_hbm.at[ix], rw, sg).wait()

        def transpose_unit(rw, tr):
            for sr in range(8):
                srev = iota + sr * 16

                @plsc.parallel_loop(0, 128, unroll=8,
                                    carry=jnp.zeros((16,), jnp.int32))
                def _(bc, bcv):
                    val = rw[sr * 128 + bc, :]
                    plsc.store_scatter(tr, [srev, bcv], val)
                    return bcv + 1

        def fire_stores(u, tr, ss):
            unit = u * 32 + wid
            st = unit >> 7
            bt = unit & 127
            for sr in range(8):
                s = st * 8 + sr
                for et in range(2):
                    pltpu.make_async_copy(
                        tr.at[pl.ds(sr * 16 + et * 8, 8), pl.ds(0, 128)],
                        out_hbm.at[s, et, bt], ss).start()

        def drain_stores(rw, ss):
            pltpu.make_async_copy(tbl_hbm.at[pl.ds(0, 1024)], rw, ss).wait()

        stage_and_fire(0, ix0, rw0, sg0)

        def body(g, _):
            u0, u1 = 2 * g, 2 * g + 1
            stage_and_fire(u1, ix1, rw1, sg1)
            wait_gather(ix0, rw0, sg0)

            @pl.when(g > 0)
            def _():
                drain_stores(rw1, ss0)
            transpose_unit(rw0, tr0)
            fire_stores(u0, tr0, ss0)

            @pl.when(g < NPAIR - 1)
            def _():
                stage_and_fire(u0 + 2, ix0, rw0, sg0)
            wait_gather(ix1, rw1, sg1)

            @pl.when(g > 0)
            def _():
                drain_stores(rw0, ss1)
            transpose_unit(rw1, tr1)
            fire_stores(u1, tr1, ss1)
            return 0

        lax.fori_loop(0, NPAIR, body, 0)
        drain_stores(rw0, ss0)
        drain_stores(rw1, ss1)

    return k(x4, tbl_rm)


def kernel(x, table):
    x = x.astype(jnp.int32)
    tab_t = table.T
    tail_lin = table[VMAIN:, :].reshape(8, 128)
    x4 = (x.T.reshape(25, 8, 128, 128)
          .transpose(0, 2, 1, 3).reshape(25, 128, 1024))
    tbl_lin = _transpose_call(tab_t, tail_lin)
    tbl_rm = tbl_lin.reshape(VOCAB, D)
    o5 = _gather_call(x4, tbl_rm)
    return o5.transpose(2, 4, 0, 1, 3).reshape(BATCH, SEQ, D)


# speedup vs baseline: 13.3428x; 1.0035x over previous
import functools

import jax
import jax.numpy as jnp
from jax import lax
from jax.experimental import pallas as pl
from jax.experimental.pallas import tpu as pltpu
from jax.experimental.pallas import tpu_sc as plsc

VOCAB = 1000000
D = 16
BATCH = 16384
SEQ = 200

VMAIN = 999936
W1 = 1536
NCH1 = VMAIN // W1
RB1 = W1 * D // 128
EXTRA1 = NCH1 - 32 * (NCH1 // 32)

NUNIT = (BATCH // 128) * (SEQ // 8)
UPT = NUNIT // 32
NPAIR = UPT // 2


def _mesh():
    return plsc.VectorSubcoreMesh(core_axis_name="c", subcore_axis_name="s")


def _transpose_call(tab_t, tail_lin):
    @functools.partial(
        pl.kernel,
        mesh=_mesh(),
        out_type=jax.ShapeDtypeStruct((125000, 128), jnp.float32),
        scratch_types=[
            pltpu.VMEM((16, W1 + 1), jnp.float32),
            pltpu.VMEM((16, W1 + 1), jnp.float32),
            pltpu.VMEM((RB1, 128), jnp.float32),
            pltpu.VMEM((RB1, 128), jnp.float32),
            pltpu.VMEM((8, 128), jnp.float32),
            pltpu.SemaphoreType.DMA,
            pltpu.SemaphoreType.DMA,
            pltpu.SemaphoreType.DMA,
            pltpu.SemaphoreType.DMA,
        ],
        compiler_params=pltpu.CompilerParams(
            use_tc_tiling_on_sc=True, needs_layout_passes=False),
    )
    def k(tab_hbm, tail_hbm, out_hbm, in0, in1, ob0, ob1, tbuf,
          si0, si1, so0, so1):
        wid = lax.axis_index("s") * 2 + lax.axis_index("c")
        n = jnp.where(wid < EXTRA1, NCH1 // 32 + 1, NCH1 // 32)
        iota = lax.iota(jnp.int32, 16)

        def in_descs(c, buf, sem):
            off = c * W1
            d1 = pltpu.make_async_copy(
                tab_hbm.at[pl.ds(0, 8), pl.ds(off, W1)],
                buf.at[pl.ds(0, 8), pl.ds(0, W1)], sem)
            d2 = pltpu.make_async_copy(
                tab_hbm.at[pl.ds(8, 8), pl.ds(off, W1)],
                buf.at[pl.ds(8, 8), pl.ds(0, W1)], sem)
            return d1, d2

        def start_in(c, buf, sem):
            d1, d2 = in_descs(c, buf, sem)
            d1.start()
            d2.start()

        def wait_in(c, buf, sem):
            d1, d2 = in_descs(c, buf, sem)
            d1.wait()
            d2.wait()

        def transpose_chunk(buf, ob):
            @plsc.parallel_loop(0, W1, unroll=16,
                                carry=jnp.zeros((16,), jnp.int32))
            def _(v, colv):
                val = plsc.load_gather(buf, [iota, colv])
                ob[v >> 3, pl.ds((v & 7) * 16, 16)] = val
                return colv + 1

        def out_desc(c, ob, sem):
            return pltpu.make_async_copy(
                ob, out_hbm.at[pl.ds(c * RB1, RB1)], sem)

        def do_chunk(i, c, buf, ob, sem_o):
            wait_in(c, buf, si0 if buf is in0 else si1)
            @pl.when(i >= 2)
            def _():
                out_desc(0, ob, sem_o).wait()
            transpose_chunk(buf, ob)
            out_desc(c, ob, sem_o).start()

        start_in(wid, in0, si0)

        def body(g, _):
            i0, i1 = 2 * g, 2 * g + 1
            c0 = wid + 32 * i0
            c1 = wid + 32 * i1

            @pl.when(i1 < n)
            def _():
                start_in(c1, in1, si1)

            @pl.when(i0 < n)
            def _():
                do_chunk(i0, c0, in0, ob0, so0)

            @pl.when(i0 + 2 < n)
            def _():
                start_in(c0 + 64, in0, si0)

            @pl.when(i1 < n)
            def _():
                do_chunk(i1, c1, in1, ob1, so1)
            return 0

        lax.fori_loop(0, (NCH1 // 32 + 2) // 2, body, 0)
        out_desc(0, ob0, so0).wait()
        out_desc(0, ob1, so1).wait()

        @pl.when(wid == 31)
        def _():
            pltpu.sync_copy(tail_hbm, tbuf)
            pltpu.sync_copy(tbuf, out_hbm.at[pl.ds(VMAIN * D // 128, 8)])

    return k(tab_t, tail_lin)


def _gather_call(x4, tbl_rm):
    @functools.partial(
        pl.kernel,
        mesh=_mesh(),
        out_type=jax.ShapeDtypeStruct((200, 2, 128, 8, 128), jnp.float32),
        scratch_types=[
            pltpu.VMEM((1024,), jnp.int32),
            pltpu.VMEM((1024,), jnp.int32),
            pltpu.VMEM((1024, 16), jnp.float32),
            pltpu.VMEM((1024, 16), jnp.float32),
            pltpu.VMEM((8, 2, 1, 8, 129), jnp.float32),
            pltpu.VMEM((8, 2, 1, 8, 129), jnp.float32),
            pltpu.SemaphoreType.DMA,
            pltpu.SemaphoreType.DMA,
            pltpu.SemaphoreType.DMA,
            pltpu.SemaphoreType.DMA,
        ],
        compiler_params=pltpu.CompilerParams(
            use_tc_tiling_on_sc=False, needs_layout_passes=False),
    )
    def k(x4_hbm, tbl_hbm, out_hbm, ix0, ix1, rw0, rw1, tr0, tr1,
          sg0, sg1, ss0, ss1):
        wid = lax.axis_index("s") * 2 + lax.axis_index("c")
        iota = lax.iota(jnp.int32, 16)

        def stage_and_fire(u, ix, rw, sg):
            unit = u * 32 + wid
            st = unit >> 7
            bt = unit & 127
            pltpu.sync_copy(x4_hbm.at[st, bt], ix)
            pltpu.make_async_copy(tbl_hbm.at[ix], rw, sg).start()

        def wait_gather(ix, rw, sg):
            pltpu.make_async_copy(tbl_hbm.at[ix], rw, sg).wait()

        e8 = iota >> 3
        em = iota & 7
        zero16 = jnp.zeros((16,), jnp.int32)

        def transpose_unit(rw, tr):
            for sr in range(8):
                srv = jnp.full((16,), sr, jnp.int32)

                @plsc.parallel_loop(0, 128, unroll=8,
                                    carry=jnp.zeros((16,), jnp.int32))
                def _(bc, bcv):
                    val = rw[sr * 128 + bc, :]
                    plsc.store_scatter(tr, [srv, e8, zero16, em, bcv], val)
                    return bcv + 1

        def fire_stores(u, tr, ss):
            unit = u * 32 + wid
            st = unit >> 7
            bt = unit & 127
            pltpu.make_async_copy(
                tr.at[:, :, :, :, pl.ds(0, 128)],
                out_hbm.at[pl.ds(st * 8, 8), :, pl.ds(bt, 1)],
                ss).start()

        def drain_stores(rw, ss):
            pltpu.make_async_copy(tbl_hbm.at[pl.ds(0, 1024)], rw, ss).wait()

        stage_and_fire(0, ix0, rw0, sg0)

        def body(g, _):
            u0, u1 = 2 * g, 2 * g + 1
            stage_and_fire(u1, ix1, rw1, sg1)
            wait_gather(ix0, rw0, sg0)

            @pl.when(g > 0)
            def _():
                drain_stores(rw1, ss0)
            transpose_unit(rw0, tr0)
            fire_stores(u0, tr0, ss0)

            @pl.when(g < NPAIR - 1)
            def _():
                stage_and_fire(u0 + 2, ix0, rw0, sg0)
            wait_gather(ix1, rw1, sg1)

            @pl.when(g > 0)
            def _():
                drain_stores(rw0, ss1)
            transpose_unit(rw1, tr1)
            fire_stores(u1, tr1, ss1)
            return 0

        lax.fori_loop(0, NPAIR, body, 0)
        drain_stores(rw0, ss0)
        drain_stores(rw1, ss1)

    return k(x4, tbl_rm)


def kernel(x, table):
    x = x.astype(jnp.int32)
    tab_t = table.T
    tail_lin = table[VMAIN:, :].reshape(8, 128)
    x4 = (x.T.reshape(25, 8, 128, 128)
          .transpose(0, 2, 1, 3).reshape(25, 128, 1024))
    tbl_lin = _transpose_call(tab_t, tail_lin)
    tbl_rm = tbl_lin.reshape(VOCAB, D)
    o5 = _gather_call(x4, tbl_rm)
    return o5.transpose(2, 4, 0, 1, 3).reshape(BATCH, SEQ, D)
